# Initial kernel scaffold; baseline (speedup 1.0000x reference)
import jax
import jax.numpy as jnp
from jax.experimental import pallas as pl


def kernel(k, u, memory_knowledge, memory_understanding, w_w, w_u, w_lu, beta_param):
    raise NotImplementedError("write your pallas kernel here")


# speedup vs baseline: 4.8999x; 4.8999x over previous
import functools

import jax
import jax.numpy as jnp
from jax.experimental import pallas as pl
from jax.experimental.pallas import tpu as pltpu

_D = 8192
_CB = 512
_NBLK = _D // _CB


def _flash_body(k_ref, wu_ref, mk_ref, out_ref, acc_ref, m_ref, l_ref, nk_ref,
                mi_ref):
    j = pl.program_id(0)

    @pl.when(j == 0)
    def _init():
        kv = k_ref[...]
        nk_ref[0, 0] = jnp.sqrt(jnp.sum(kv * kv))
        wu = wu_ref[...]
        mn = jnp.min(wu)
        ids = jax.lax.broadcasted_iota(jnp.int32, (1, _D), 1)
        mi_ref[0, 0] = jnp.min(jnp.where(wu == mn, ids, _D))
        m_ref[0, 0] = -jnp.inf
        l_ref[0, 0] = 0.0

    blk = mk_ref[...]
    kv = k_ref[...]
    cs = jnp.sum(blk * blk, axis=0, keepdims=True)
    dt = jax.lax.dot_general(kv, blk, (((1,), (0,)), ((), ())),
                             preferred_element_type=jnp.float32)
    sim = dt / (nk_ref[0, 0] * jnp.sqrt(cs))
    m_old = m_ref[0, 0]
    m_new = jnp.maximum(m_old, jnp.max(sim))
    p = jnp.exp(sim - m_new)
    scale = jnp.exp(m_old - m_new)
    l_ref[0, 0] = l_ref[0, 0] * scale + jnp.sum(p)
    m_ref[0, 0] = m_new
    col = j * _CB + jax.lax.broadcasted_iota(jnp.int32, (1, _CB), 1)
    pz = jnp.where(col == mi_ref[0, 0], 0.0, p)
    contrib = jax.lax.dot_general(blk, pz, (((1,), (1,)), ((), ())),
                                  preferred_element_type=jnp.float32)

    @pl.when(j == 0)
    def _first():
        acc_ref[...] = contrib

    @pl.when(j > 0)
    def _rest():
        acc_ref[...] = acc_ref[...] * scale + contrib

    @pl.when(j == _NBLK - 1)
    def _fin():
        out_ref[...] = acc_ref[...] / l_ref[0, 0]


def kernel(k, u, memory_knowledge, memory_understanding, w_w, w_u, w_lu,
           beta_param):
    k2 = k.reshape(1, _D)
    wu2 = w_u.reshape(1, _D)
    out = pl.pallas_call(
        _flash_body,
        grid=(_NBLK,),
        in_specs=[
            pl.BlockSpec((1, _D), lambda j: (0, 0)),
            pl.BlockSpec((1, _D), lambda j: (0, 0)),
            pl.BlockSpec((_D, _CB), lambda j: (0, j)),
        ],
        out_specs=pl.BlockSpec((_D, 1), lambda j: (0, 0)),
        out_shape=jax.ShapeDtypeStruct((_D, 1), jnp.float32),
        scratch_shapes=[
            pltpu.VMEM((_D, 1), jnp.float32),
            pltpu.SMEM((1, 1), jnp.float32),
            pltpu.SMEM((1, 1), jnp.float32),
            pltpu.SMEM((1, 1), jnp.float32),
            pltpu.SMEM((1, 1), jnp.int32),
        ],
        compiler_params=pltpu.CompilerParams(
            dimension_semantics=("arbitrary",),
        ),
    )(k2, wu2, memory_knowledge)
    return out.reshape(1, _D)
